# baseline (speedup 1.0000x reference)
import jax
import jax.numpy as jnp
from jax.experimental import pallas as pl
from jax.experimental.pallas import tpu as pltpu

_H = 32
_NH = 12
_S = _H * _H
_M = 2 * _H - 1
_NP = 4
_BP = _S // _NP
_BPI = _BP // _H


def _fused_body(rp_ref, x_ref, o_ref, bias_ref):
    pb = pl.program_id(2)

    @pl.when(pl.program_id(1) == 0)
    def _build_bias_quarter():
        rp64 = jnp.pad(rp_ref[0], ((0, 1), (0, 1)))
        r0 = pb * _BP
        r = r0 + jax.lax.broadcasted_iota(jnp.int32, (_BP, 64), 0)
        a = jax.lax.broadcasted_iota(jnp.int32, (_BP, 64), 1)
        oi = (a == (r // _H - r % _H + (_H - 1))).astype(jnp.float32)
        c = jax.lax.broadcasted_iota(jnp.int32, (64, _S), 1)
        b = jax.lax.broadcasted_iota(jnp.int32, (64, _S), 0)
        ojt = (b == (c // _H - c % _H + (_H - 1))).astype(jnp.float32)
        t1 = jnp.dot(oi, rp64, preferred_element_type=jnp.float32)
        t2 = jnp.dot(t1, ojt, preferred_element_type=jnp.float32)
        t4 = t2.reshape(_BPI, _H, _H, _H).transpose(0, 2, 1, 3)
        bias_ref[pl.ds(pb * _BP, _BP), :] = t4.reshape(_BP, _S)

    o_ref[0, 0] = x_ref[0, 0] + bias_ref[pl.ds(pb * _BP, _BP), :]


def kernel(x, relative_pos):
    return pl.pallas_call(
        _fused_body,
        grid=(_NH, x.shape[0], _NP),
        in_specs=[
            pl.BlockSpec((1, _M, _M), lambda h, b, p: (h, 0, 0)),
            pl.BlockSpec((1, 1, _BP, _S), lambda h, b, p: (b, h, p, 0)),
        ],
        out_specs=pl.BlockSpec((1, 1, _BP, _S), lambda h, b, p: (b, h, p, 0)),
        out_shape=jax.ShapeDtypeStruct(x.shape, x.dtype),
        scratch_shapes=[pltpu.VMEM((_S, _S), jnp.float32)],
    )(relative_pos, x)
